# Initial kernel scaffold; baseline (speedup 1.0000x reference)
import jax
import jax.numpy as jnp
from jax.experimental import pallas as pl


def kernel(hn, pos, batch, edge_index, r, W_msg1, W_upd1, W_msg2, W_upd2, W_lin, W_tp, W_sh, W_enc1, W_enc2):
    raise NotImplementedError("write your pallas kernel here")


# speedup vs baseline: 1.7509x; 1.7509x over previous
import functools

import jax
import jax.numpy as jnp
from jax import lax
from jax.experimental import pallas as pl
from jax.experimental.pallas import tpu as pltpu
from jax.experimental.pallas import tpu_sc as plsc

N = 10000
NP = 10240
E = 160000
EP = 163840
D = 128
K = N // 2
KP = 5120
SORT = 16384
NW = 32
NCH = 40
EPW = EP // NW
TPW = KP // NW
RPT = NP // 16

@functools.cache
def _sc_mesh():
    return plsc.VectorSubcoreMesh(core_axis_name="c", subcore_axis_name="s")


def _gelu(x):
    return jax.nn.gelu(x)


def _rsum128(x):
    acc = x[:, 0:8]
    for t in range(1, 16):
        acc = acc + x[:, 8 * t:8 * t + 8]
    a4 = acc[:, 0:4] + acc[:, 4:8]
    a2 = a4[:, 0:2] + a4[:, 2:4]
    return a2[:, 0:1] + a2[:, 1:2]


def _rsum16(x):
    a8 = x[:, 0:8] + x[:, 8:16]
    a4 = a8[:, 0:4] + a8[:, 4:8]
    a2 = a4[:, 0:2] + a4[:, 2:4]
    return a2[:, 0:1] + a2[:, 1:2]


def _rmsnorm(x):
    return x * lax.rsqrt(_rsum128(x * x) / 128.0 + 1e-6)


def _mm_gelu_body(h_ref, w_ref, o_ref):
    o_ref[...] = _gelu(jnp.dot(h_ref[...], w_ref[...],
                               preferred_element_type=jnp.float32))


def _mm_gelu(h, w):
    br = 512
    return pl.pallas_call(
        _mm_gelu_body,
        grid=(NP // br,),
        in_specs=[pl.BlockSpec((br, D), lambda i: (i, 0)),
                  pl.BlockSpec((D, D), lambda i: (0, 0))],
        out_specs=pl.BlockSpec((br, D), lambda i: (i, 0)),
        out_shape=jax.ShapeDtypeStruct((NP, D), jnp.float32),
    )(h, w)


def _upd_msg_body(h_ref, p_ref, wu_ref, wm_ref, h1_ref, gh_ref):
    x = h_ref[...] + p_ref[...]
    h1 = _rmsnorm(_gelu(jnp.dot(x, wu_ref[...],
                                preferred_element_type=jnp.float32)))
    h1_ref[...] = h1
    gh_ref[...] = _gelu(jnp.dot(h1, wm_ref[...],
                                preferred_element_type=jnp.float32))


def _upd_msg(h, agg, wu, wm):
    br = 512
    return pl.pallas_call(
        _upd_msg_body,
        grid=(NP // br,),
        in_specs=[pl.BlockSpec((br, D), lambda i: (i, 0)),
                  pl.BlockSpec((br, D), lambda i: (i, 0)),
                  pl.BlockSpec((D, D), lambda i: (0, 0)),
                  pl.BlockSpec((D, D), lambda i: (0, 0))],
        out_specs=[pl.BlockSpec((br, D), lambda i: (i, 0)),
                   pl.BlockSpec((br, D), lambda i: (i, 0))],
        out_shape=[jax.ShapeDtypeStruct((NP, D), jnp.float32),
                   jax.ShapeDtypeStruct((NP, D), jnp.float32)],
    )(h, agg, wu, wm)


def _upd_score_body(h_ref, p_ref, wu_ref, wl_ref, wtp_ref, h2_ref, nk_ref):
    br = h_ref.shape[0]
    x = h_ref[...] + p_ref[...]
    h2 = _rmsnorm(_gelu(jnp.dot(x, wu_ref[...],
                                preferred_element_type=jnp.float32)))
    h2_ref[...] = h2
    sh = jnp.dot(h2, wl_ref[...], preferred_element_type=jnp.float32)
    t = jnp.dot(sh, wtp_ref[...], preferred_element_type=jnp.float32)
    score = _rsum16(t * sh)
    gidx = (pl.program_id(0) * br
            + lax.broadcasted_iota(jnp.int32, (br, 1), 0))
    nk_ref[...] = jnp.where(gidx < N, -score, jnp.inf)


def _upd_score(h, agg, wu, wl, wtp):
    br = 512
    return pl.pallas_call(
        _upd_score_body,
        grid=(NP // br,),
        in_specs=[pl.BlockSpec((br, D), lambda i: (i, 0)),
                  pl.BlockSpec((br, D), lambda i: (i, 0)),
                  pl.BlockSpec((D, D), lambda i: (0, 0)),
                  pl.BlockSpec((D, 16), lambda i: (0, 0)),
                  pl.BlockSpec((16, 16), lambda i: (0, 0))],
        out_specs=[pl.BlockSpec((br, D), lambda i: (i, 0)),
                   pl.BlockSpec((br, 1), lambda i: (i, 0))],
        out_shape=[jax.ShapeDtypeStruct((NP, D), jnp.float32),
                   jax.ShapeDtypeStruct((NP, 1), jnp.float32)],
    )(h, agg, wu, wl, wtp)


def _cx_stage(keys, idx, j, k, row, col, flat):
    if j >= 128:
        sh = j // 128
        bit = (row & sh) != 0
        up_k = jnp.concatenate([keys[sh:], keys[:sh]], axis=0)
        dn_k = jnp.concatenate([keys[128 - sh:], keys[:128 - sh]], axis=0)
        up_i = jnp.concatenate([idx[sh:], idx[:sh]], axis=0)
        dn_i = jnp.concatenate([idx[128 - sh:], idx[:128 - sh]], axis=0)
    else:
        bit = (col & j) != 0
        up_k = pltpu.roll(keys, 128 - j, 1)
        dn_k = pltpu.roll(keys, j, 1)
        up_i = pltpu.roll(idx, 128 - j, 1)
        dn_i = pltpu.roll(idx, j, 1)
    pk = jnp.where(bit, dn_k, up_k)
    pi = jnp.where(bit, dn_i, up_i)
    desc = (flat & k) != 0
    gt = (keys > pk) | ((keys == pk) & (idx > pi))
    lower = jnp.logical_not(bit)
    wantmax = lower == desc
    take_self = gt == wantmax
    return jnp.where(take_self, keys, pk), jnp.where(take_self, idx, pi)


def _sort_body(nk_ref, ks_ref, is_ref):
    row = lax.broadcasted_iota(jnp.int32, (128, 128), 0)
    col = lax.broadcasted_iota(jnp.int32, (128, 128), 1)
    flat = row * 128 + col
    keys = nk_ref[...]
    idx = flat
    for p in range(1, 15):
        k = 1 << p
        j = k // 2
        while j >= 1:
            keys, idx = _cx_stage(keys, idx, j, k, row, col, flat)
            j //= 2
    ks_ref[...] = keys
    is_ref[...] = idx


def _sort(nk128):
    return pl.pallas_call(
        _sort_body,
        out_shape=[jax.ShapeDtypeStruct((128, 128), jnp.float32),
                   jax.ShapeDtypeStruct((128, 128), jnp.int32)],
    )(nk128)


def _select_body(ks_ref, is_ref, nk_ref, sel_ref, gate_ref, it_ref):
    cm = lax.broadcasted_iota(jnp.int32, (1, 128), 1) == (K - 1) % 128
    krow = ks_ref[(K - 1) // 128:(K - 1) // 128 + 1, :]
    irow = is_ref[(K - 1) // 128:(K - 1) // 128 + 1, :]
    t = jnp.max(jnp.where(cm, krow, -jnp.inf))
    ib = jnp.max(jnp.where(cm, irow, -(2 ** 31 - 1)))
    nk = nk_ref[...]
    gi = (lax.broadcasted_iota(jnp.int32, (NP // 128, 128), 0) * 128
          + lax.broadcasted_iota(jnp.int32, (NP // 128, 128), 1))
    sel_ref[...] = jnp.where((nk < t) | ((nk == t) & (gi <= ib)), 1.0, 0.0)
    gate_ref[...] = jax.nn.sigmoid(-ks_ref[0:KP // 128, :])
    it_ref[...] = is_ref[0:KP // 128, :]


def _select(keys_s, idx_s, nk2d):
    return pl.pallas_call(
        _select_body,
        out_shape=[jax.ShapeDtypeStruct((NP // 128, 128), jnp.float32),
                   jax.ShapeDtypeStruct((KP // 128, 128), jnp.float32),
                   jax.ShapeDtypeStruct((KP // 128, 128), jnp.int32)],
    )(keys_s, idx_s, nk2d)


def _gate_mul_body(a_ref, b_ref, o_ref):
    o_ref[...] = a_ref[...] * b_ref[...]


def _gate_mul(a, b):
    br = 512
    return pl.pallas_call(
        _gate_mul_body,
        grid=(KP // br,),
        in_specs=[pl.BlockSpec((br, D), lambda i: (i, 0)),
                  pl.BlockSpec((br, D), lambda i: (i, 0))],
        out_specs=pl.BlockSpec((br, D), lambda i: (i, 0)),
        out_shape=jax.ShapeDtypeStruct((KP, D), jnp.float32),
    )(a, b)


def _edge_enc_body(rx_ref, ry_ref, rz_ref, ms_ref, wsh_ref, w1_ref, w2_ref,
                   rr_ref, he_ref):
    rx = rx_ref[0]
    ry = ry_ref[0]
    rz = rz_ref[0]
    ms = ms_ref[0]
    dist = jnp.sqrt(rx * rx + ry * ry + rz * rz)
    rr = rr_ref[0, 0]
    em = ms * jnp.where(dist < rr, 1.0, 0.0)
    feats = jnp.concatenate([rx, ry, rz, dist], axis=0)
    fe_t = jnp.tanh(jnp.dot(wsh_ref[...], feats,
                            preferred_element_type=jnp.float32)) * em
    g_t = jnp.dot(w1_ref[...], fe_t,
                  preferred_element_type=jnp.float32)
    he_t = jnp.dot(w2_ref[...], jax.nn.sigmoid(g_t) * g_t,
                   preferred_element_type=jnp.float32)
    he_ref[...] = he_t.T


def _edge_enc(rx, ry, rz, ms, wsh_t, w1_t, w2_t, rr):
    nb = EP // 128
    vec = pl.BlockSpec((1, 1, 128), lambda i: (i, 0, 0))
    return pl.pallas_call(
        _edge_enc_body,
        grid=(nb,),
        in_specs=[vec, vec, vec, vec,
                  pl.BlockSpec((16, 4), lambda i: (0, 0)),
                  pl.BlockSpec((D, 16), lambda i: (0, 0)),
                  pl.BlockSpec((D, D), lambda i: (0, 0)),
                  pl.BlockSpec(memory_space=pltpu.SMEM)],
        out_specs=pl.BlockSpec((128, D), lambda i: (i, 0)),
        out_shape=jax.ShapeDtypeStruct((EP, D), jnp.float32),
    )(rx.reshape(nb, 1, 128), ry.reshape(nb, 1, 128), rz.reshape(nb, 1, 128),
      ms.reshape(nb, 1, 128), wsh_t, w1_t, w2_t, rr)


LCAP = 8192
NPT = NP // NW

_BOUNDS = tuple(
    h * 80000 + b
    for h in (0, 1)
    for b in [5040 * k for k in range(1, 15)] + [75360]
) + (80000,)


def _sc_prep(dsta, srca):
    @functools.partial(
        pl.kernel,
        out_type=[jax.ShapeDtypeStruct((NW, NPT), jnp.int32),
                  jax.ShapeDtypeStruct((NW, LCAP), jnp.int32),
                  jax.ShapeDtypeStruct((NW, LCAP), jnp.int32)],
        mesh=_sc_mesh(),
        compiler_params=pltpu.CompilerParams(needs_layout_passes=False),
        scratch_types=[
            pltpu.VMEM((2048,), jnp.int32),
            pltpu.VMEM((2048,), jnp.int32),
            pltpu.VMEM((LCAP + 32,), jnp.int32),
            pltpu.VMEM((LCAP + 32,), jnp.int32),
            pltpu.VMEM((NPT,), jnp.int32),
            pltpu.SMEM((NPT,), jnp.int32),
            pltpu.SMEM((1,), jnp.int32),
        ],
    )
    def body(dst_hbm, src_hbm, cnts_hbm, lsrc_hbm, ldst_hbm,
             dchunk, schunk, lsrcv, ldstv, countv, count_s, cnt_ref):
        c = lax.axis_index("c")
        s = lax.axis_index("s")
        w = s * 2 + c
        base = w * NPT

        @pl.loop(0, (LCAP + 32) // 16)
        def _z(i):
            lsrcv[pl.ds(i * 16, 16)] = jnp.zeros((16,), jnp.int32)
            ldstv[pl.ds(i * 16, 16)] = jnp.zeros((16,), jnp.int32)

        @pl.loop(0, NPT)
        def _z2(i):
            count_s[i] = 0

        cnt_ref[0] = 0
        lane = lax.iota(jnp.int32, 16)

        @pl.loop(0, EP // 2048)
        def _ch(ch):
            pltpu.sync_copy(dst_hbm.at[pl.ds(ch * 2048, 2048)], dchunk)
            pltpu.sync_copy(src_hbm.at[pl.ds(ch * 2048, 2048)], schunk)

            @pl.loop(0, 128)
            def _st(i):
                d16 = dchunk[pl.ds(i * 16, 16)]
                s16 = schunk[pl.ds(i * 16, 16)]
                mask = (d16 >= base) & (d16 < base + NPT)
                cum = jnp.where(mask, 1, 0).astype(jnp.int32)
                for sft in (1, 2, 4, 8):
                    shv = cum.at[jnp.maximum(lane - sft, 0)].get(
                        mode="promise_in_bounds")
                    cum = cum + jnp.where(lane >= sft, shv, 0)
                cc = cnt_ref[0]

                @pl.when(cc <= LCAP - 16)
                def _():
                    idxs = jnp.where(mask, cc + cum - 1, LCAP + 16)
                    plsc.store_scatter(lsrcv, [idxs], s16)
                    plsc.store_scatter(ldstv, [idxs], d16 - base)
                    cnt_ref[0] = cc + cum[15]

        @pl.loop(0, cnt_ref[0])
        def _cnt(i):
            nl = ldstv[pl.ds(i, 16)][0]
            count_s[nl] = count_s[nl] + 1

        @pl.loop(0, NPT // 16)
        def _pub(i):
            v = jnp.zeros((16,), jnp.int32)
            for t in range(16):
                v = jnp.where(lane == t, count_s[i * 16 + t], v)
            countv[pl.ds(i * 16, 16)] = v

        pltpu.sync_copy(countv, cnts_hbm.at[w])
        pltpu.sync_copy(lsrcv.at[pl.ds(0, LCAP)], lsrc_hbm.at[w])
        pltpu.sync_copy(ldstv.at[pl.ds(0, LCAP)], ldst_hbm.at[w])

    return body(dsta, srca)


def _sc_agg(gh, lsrc, ldst, cap0, tcnt8):
    @functools.partial(
        pl.kernel,
        out_type=jax.ShapeDtypeStruct((NP, D), jnp.float32),
        mesh=_sc_mesh(),
        compiler_params=pltpu.CompilerParams(needs_layout_passes=False),
        scratch_types=[
            pltpu.VMEM((LCAP + 16,), jnp.int32),
            pltpu.VMEM((LCAP + 16,), jnp.int32),
            pltpu.VMEM((NPT,), jnp.int32),
            pltpu.VMEM((16,), jnp.int32),
            pltpu.VMEM((NPT, D), jnp.float32),
            pltpu.VMEM((NPT, D), jnp.float32),
            pltpu.VMEM((128, D), jnp.float32),
            pltpu.SMEM((NPT,), jnp.int32),
            pltpu.SemaphoreType.DMA,
        ],
    )
    def body(gh_hbm, lsrc_hbm, ldst_hbm, cap_hbm, tc_hbm, out_hbm,
             lsrcv, ldstv, capv, cntv, P, T, rows, cap_s, sem):
        c = lax.axis_index("c")
        s = lax.axis_index("s")
        w = s * 2 + c
        base = w * NPT
        pltpu.sync_copy(lsrc_hbm.at[w], lsrcv.at[pl.ds(0, LCAP)])
        pltpu.sync_copy(ldst_hbm.at[w], ldstv.at[pl.ds(0, LCAP)])
        pltpu.sync_copy(cap_hbm.at[pl.ds(base, NPT)], capv)
        pltpu.sync_copy(tc_hbm.at[w], cntv)

        @pl.loop(0, NPT // 16)
        def _cs(i):
            v = capv[pl.ds(i * 16, 16)]
            for t in range(16):
                cap_s[i * 16 + t] = v[t]

        @pl.loop(0, NPT)
        def _z(r):
            for f in range(8):
                sl = pl.ds(f * 16, 16)
                P[r, sl] = jnp.zeros((16,), jnp.float32)
                T[r, sl] = jnp.zeros((16,), jnp.float32)

        cnt = cntv[...][0]
        nfull = (cnt + 127) // 128

        @pl.loop(0, nfull)
        def _blk(j):
            pltpu.async_copy(gh_hbm.at[lsrcv.at[pl.ds(j * 128, 128)]],
                             rows, sem).wait()

            @pl.loop(0, 128)
            def _e(tl):
                idx = j * 128 + tl

                @pl.when(idx < cnt)
                def _():
                    nl = ldstv[pl.ds(idx, 16)][0]
                    cc = cap_s[nl]

                    @pl.when(cc == 0)
                    def _flush():
                        for f in range(8):
                            sl = pl.ds(f * 16, 16)
                            T[nl, sl] = T[nl, sl] + P[nl, sl]
                            P[nl, sl] = jnp.zeros((16,), jnp.float32)

                    cap_s[nl] = jnp.where(cc == 0, (1 << 24) - 1, cc - 1)
                    for f in range(8):
                        sl = pl.ds(f * 16, 16)
                        P[nl, sl] = P[nl, sl] + rows[tl, sl]

        @pl.loop(0, NPT)
        def _fin(r):
            for f in range(8):
                sl = pl.ds(f * 16, 16)
                T[r, sl] = T[r, sl] + P[r, sl]

        pltpu.sync_copy(T, out_hbm.at[pl.ds(base, NPT)])

    return body(gh, lsrc, ldst, cap0, tcnt8)


def _sc_subgather(h2, idxt, posx, posy, posz, batch_p):
    @functools.partial(
        pl.kernel,
        out_type=[jax.ShapeDtypeStruct((KP, D), jnp.float32),
                  jax.ShapeDtypeStruct((KP,), jnp.float32),
                  jax.ShapeDtypeStruct((KP,), jnp.float32),
                  jax.ShapeDtypeStruct((KP,), jnp.float32),
                  jax.ShapeDtypeStruct((KP,), jnp.int32)],
        mesh=_sc_mesh(),
        compiler_params=pltpu.CompilerParams(needs_layout_passes=False),
        scratch_types=[
            pltpu.VMEM((TPW,), jnp.int32),
            pltpu.VMEM((NP,), jnp.float32),
            pltpu.VMEM((NP,), jnp.float32),
            pltpu.VMEM((NP,), jnp.float32),
            pltpu.VMEM((NP,), jnp.int32),
            pltpu.VMEM((TPW,), jnp.float32),
            pltpu.VMEM((TPW,), jnp.float32),
            pltpu.VMEM((TPW,), jnp.float32),
            pltpu.VMEM((TPW,), jnp.int32),
            pltpu.VMEM((TPW // 2, D), jnp.float32),
            pltpu.SemaphoreType.DMA,
        ],
    )
    def body(h2_hbm, it_hbm, px_hbm, py_hbm, pz_hbm, b_hbm,
             hs_hbm, ox_hbm, oy_hbm, oz_hbm, ob_hbm,
             idxv, pxv, pyv, pzv, bv, oxv, oyv, ozv, obv, rows, sem):
        c = lax.axis_index("c")
        s = lax.axis_index("s")
        wid = s * 2 + c
        base = wid * TPW
        pltpu.sync_copy(it_hbm.at[pl.ds(base, TPW)], idxv)
        pltpu.sync_copy(px_hbm, pxv)
        pltpu.sync_copy(py_hbm, pyv)
        pltpu.sync_copy(pz_hbm, pzv)
        pltpu.sync_copy(b_hbm, bv)

        @pl.loop(0, TPW // 16)
        def _lanes(t):
            i16 = idxv[pl.ds(t * 16, 16)]
            oxv[pl.ds(t * 16, 16)] = plsc.load_gather(pxv, [i16])
            oyv[pl.ds(t * 16, 16)] = plsc.load_gather(pyv, [i16])
            ozv[pl.ds(t * 16, 16)] = plsc.load_gather(pzv, [i16])
            obv[pl.ds(t * 16, 16)] = plsc.load_gather(bv, [i16])

        @pl.loop(0, 2)
        def _rows(j):
            pltpu.async_copy(h2_hbm.at[idxv.at[pl.ds(j * (TPW // 2), TPW // 2)]],
                             rows, sem).wait()
            pltpu.sync_copy(rows, hs_hbm.at[pl.ds(base + j * (TPW // 2),
                                                  TPW // 2)])

        pltpu.sync_copy(oxv, ox_hbm.at[pl.ds(base, TPW)])
        pltpu.sync_copy(oyv, oy_hbm.at[pl.ds(base, TPW)])
        pltpu.sync_copy(ozv, oz_hbm.at[pl.ds(base, TPW)])
        pltpu.sync_copy(obv, ob_hbm.at[pl.ds(base, TPW)])

    return body(h2, idxt, posx, posy, posz, batch_p)


def _sc_edgegeom(src2, dst2, posx, posy, posz, sel):
    @functools.partial(
        pl.kernel,
        out_type=[jax.ShapeDtypeStruct((EP,), jnp.float32),
                  jax.ShapeDtypeStruct((EP,), jnp.float32),
                  jax.ShapeDtypeStruct((EP,), jnp.float32),
                  jax.ShapeDtypeStruct((EP,), jnp.float32)],
        mesh=_sc_mesh(),
        compiler_params=pltpu.CompilerParams(needs_layout_passes=False),
        scratch_types=[
            pltpu.VMEM((EPW,), jnp.int32),
            pltpu.VMEM((EPW,), jnp.int32),
            pltpu.VMEM((NP,), jnp.float32),
            pltpu.VMEM((NP,), jnp.float32),
            pltpu.VMEM((NP,), jnp.float32),
            pltpu.VMEM((NP,), jnp.float32),
            pltpu.VMEM((EPW,), jnp.float32),
            pltpu.VMEM((EPW,), jnp.float32),
            pltpu.VMEM((EPW,), jnp.float32),
            pltpu.VMEM((EPW,), jnp.float32),
        ],
    )
    def body(src_hbm, dst_hbm, px_hbm, py_hbm, pz_hbm, sel_hbm,
             orx_hbm, ory_hbm, orz_hbm, oms_hbm,
             srcv, dstv, pxv, pyv, pzv, selv, orx, ory, orz, oms):
        c = lax.axis_index("c")
        s = lax.axis_index("s")
        wid = s * 2 + c
        pltpu.sync_copy(src_hbm.at[wid], srcv)
        pltpu.sync_copy(dst_hbm.at[wid], dstv)
        pltpu.sync_copy(px_hbm, pxv)
        pltpu.sync_copy(py_hbm, pyv)
        pltpu.sync_copy(pz_hbm, pzv)
        pltpu.sync_copy(sel_hbm, selv)

        @pl.loop(0, EPW // 16)
        def _lanes(t):
            s16 = srcv[pl.ds(t * 16, 16)]
            d16 = dstv[pl.ds(t * 16, 16)]
            orx[pl.ds(t * 16, 16)] = (plsc.load_gather(pxv, [d16])
                                      - plsc.load_gather(pxv, [s16]))
            ory[pl.ds(t * 16, 16)] = (plsc.load_gather(pyv, [d16])
                                      - plsc.load_gather(pyv, [s16]))
            orz[pl.ds(t * 16, 16)] = (plsc.load_gather(pzv, [d16])
                                      - plsc.load_gather(pzv, [s16]))
            oms[pl.ds(t * 16, 16)] = (plsc.load_gather(selv, [s16])
                                      * plsc.load_gather(selv, [d16]))

        pltpu.sync_copy(orx, orx_hbm.at[pl.ds(wid * EPW, EPW)])
        pltpu.sync_copy(ory, ory_hbm.at[pl.ds(wid * EPW, EPW)])
        pltpu.sync_copy(orz, orz_hbm.at[pl.ds(wid * EPW, EPW)])
        pltpu.sync_copy(oms, oms_hbm.at[pl.ds(wid * EPW, EPW)])

    return body(src2, dst2, posx, posy, posz, sel)


def kernel(hn, pos, batch, edge_index, r, W_msg1, W_upd1, W_msg2, W_upd2,
           W_lin, W_tp, W_sh, W_enc1, W_enc2):
    f32 = jnp.float32
    hn_p = jnp.pad(hn, ((0, NP - N), (0, 0)))
    posx = jnp.pad(pos[:, 0], (0, NP - N))
    posy = jnp.pad(pos[:, 1], (0, NP - N))
    posz = jnp.pad(pos[:, 2], (0, NP - N))
    batch_p = jnp.pad(batch, (0, NP - N)).astype(jnp.int32)

    src = edge_index[0].astype(jnp.int32)
    dst = edge_index[1].astype(jnp.int32)
    srcp = jnp.pad(src, (0, EP - E), constant_values=N)
    dstp = jnp.pad(dst, (0, EP - E), constant_values=N)
    src2 = srcp.reshape(NW, EPW)
    dst2 = dstp.reshape(NW, EPW)
    srca = jnp.pad(src, (0, EP - E), constant_values=0)
    dsta = jnp.pad(dst, (0, EP - E), constant_values=1 << 30)

    counts, lsrc, ldst = _sc_prep(dsta, srca)
    cflat = counts.reshape(NP)
    segstart = jnp.cumsum(cflat) - cflat
    bounds = jnp.asarray(_BOUNDS, jnp.int32)
    nb = jnp.min(jnp.where(bounds[None, :] > segstart[:, None],
                           bounds[None, :], 1 << 30), axis=1)
    cap0 = (nb - segstart).astype(jnp.int32)
    tcnt8 = jnp.broadcast_to(
        jnp.sum(counts, axis=1, dtype=jnp.int32)[:, None], (NW, 16))

    gh1 = _mm_gelu(hn_p, W_msg1)
    agg1 = _sc_agg(gh1, lsrc, ldst, cap0, tcnt8)
    h1, gh2 = _upd_msg(hn_p, agg1, W_upd1, W_msg2)

    agg2 = _sc_agg(gh2, lsrc, ldst, cap0, tcnt8)
    h2, nk = _upd_score(h1, agg2, W_upd2, W_lin, W_tp)

    nkf = nk[:, 0]
    nk128 = jnp.pad(nkf, (0, SORT - NP),
                    constant_values=jnp.inf).reshape(128, 128)
    keys_s, idx_s = _sort(nk128)
    sel2, gate2, it2 = _select(keys_s, idx_s, nkf.reshape(NP // 128, 128))
    sel = sel2.reshape(NP)
    gate = gate2.reshape(KP)
    idxt = it2.reshape(KP)

    hsub_raw, ox, oy, oz, ob = _sc_subgather(h2, idxt, posx, posy, posz,
                                             batch_p)
    rx, ry, rz, ms = _sc_edgegeom(src2, dst2, posx, posy, posz, sel)

    hn_sub = _gate_mul(hsub_raw, jnp.broadcast_to(gate[:, None], (KP, D)))[:K]
    rr = jnp.asarray(r, f32).reshape(1, 1)
    he = _edge_enc(rx, ry, rz, ms, W_sh.T, W_enc1.T, W_enc2.T, rr)[:E]
    pos_sub = jnp.stack([ox[:K], oy[:K], oz[:K]], axis=1)
    batch_sub = ob[:K]
    return hn_sub, pos_sub, batch_sub, he
